# baseline (device time: 89017 ns/iter reference)
import jax
import jax.numpy as jnp
from jax import lax
from jax.experimental import pallas as pl
from jax.experimental.pallas import tpu as pltpu

N_DEV = 4
N_HOP = N_DEV - 1
N_SUB = 4


def kernel(x, w_mat, scale_x, scale_w):
    m_per, k = x.shape
    _, n_per = w_mat.shape
    half = m_per // 2
    sub = half // N_SUB

    def body(x_ref, w_ref, sx_ref, sw_ref, out_ref,
             buf_a, buf_b, stage, w32, w16, outblk,
             stage_sem, w_sem, out_sems, *ring_sems):
        my = lax.axis_index("i")
        left = lax.rem(my + N_DEV - 1, N_DEV)
        right = lax.rem(my + 1, N_DEV)

        dirs = {
            "a": dict(buf=buf_a, peer=right, base=0,
                      origin=lambda s: lax.rem(my - s + N_DEV, N_DEV)),
            "b": dict(buf=buf_b, peer=left, base=half,
                      origin=lambda s: lax.rem(my + s, N_DEV)),
        }
        for q in range(N_SUB):
            for di, d in enumerate(("a", "b")):
                dirs[d].setdefault("send", {})[q] = ring_sems[4 * q + 2 * di]
                dirs[d].setdefault("recv", {})[q] = ring_sems[4 * q + 2 * di + 1]

        barrier = pltpu.get_barrier_semaphore()
        for nbr in (left, right):
            pl.semaphore_signal(
                barrier, inc=1,
                device_id=(nbr,), device_id_type=pl.DeviceIdType.MESH,
            )
        pl.semaphore_wait(barrier, 2)

        def make_rdma(d, h, q):
            dd = dirs[d]
            return pltpu.make_async_remote_copy(
                src_ref=dd["buf"].at[h, pl.ds(q * sub, sub), :],
                dst_ref=dd["buf"].at[h + 1, pl.ds(q * sub, sub), :],
                send_sem=dd["send"][q].at[h],
                recv_sem=dd["recv"][q].at[h],
                device_id=(dd["peer"],),
                device_id_type=pl.DeviceIdType.MESH,
            )

        rdmas = {}

        for q in range(N_SUB):
            for d in ("a", "b"):
                rows = dirs[d]["base"] + q * sub
                cp = pltpu.make_async_copy(
                    x_ref.at[pl.ds(rows, sub), :], stage, stage_sem)
                cp.start()
                cp.wait()
                dirs[d]["buf"][0, pl.ds(q * sub, sub), :] = (
                    stage[...].astype(jnp.float8_e4m3fn))
                r = make_rdma(d, 0, q)
                r.start()
                rdmas[(d, 0, q)] = r

        cp_w = pltpu.make_async_copy(w_ref, w32, w_sem)
        cp_w.start()
        cp_w.wait()
        w16[...] = w32[...].astype(jnp.bfloat16)

        scale = sx_ref[0, 0] * sw_ref[0, 0]
        out_copies = []

        def compute_block(d, s, row0, nrows):
            dd = dirs[d]
            blk = len(out_copies)
            slot = blk % 2
            if blk >= 2:
                out_copies[blk - 2].wait()
            acc = jnp.dot(
                dd["buf"][s, pl.ds(row0, nrows), :].astype(jnp.bfloat16),
                w16[...], preferred_element_type=jnp.float32)
            outblk[slot, pl.ds(0, nrows), :] = jnp.maximum(acc * scale, 0.0)
            base = dd["origin"](s) * m_per + dd["base"]
            cp = pltpu.make_async_copy(
                outblk.at[slot, pl.ds(0, nrows), :],
                out_ref.at[pl.ds(base + row0, nrows), :],
                out_sems.at[slot],
            )
            cp.start()
            out_copies.append(cp)

        def compute_slot(s):
            compute_block("a", s, 0, half)
            compute_block("b", s, 0, half)

        compute_slot(0)

        for h in range(1, N_HOP):
            for q in range(N_SUB):
                for d in ("a", "b"):
                    rdmas[(d, h - 1, q)].wait()
                    r = make_rdma(d, h, q)
                    r.start()
                    rdmas[(d, h, q)] = r
            compute_slot(h)

        for q in range(N_SUB):
            for d in ("a", "b"):
                rdmas[(d, N_HOP - 1, q)].wait()
            for d in ("a", "b"):
                compute_block(d, N_DEV - 1, q * sub, sub)

        out_copies[-2].wait()
        out_copies[-1].wait()

    return pl.pallas_call(
        body,
        out_shape=jax.ShapeDtypeStruct((N_DEV * m_per, n_per), jnp.float32),
        in_specs=[
            pl.BlockSpec(memory_space=pl.ANY),
            pl.BlockSpec(memory_space=pl.ANY),
            pl.BlockSpec(memory_space=pltpu.SMEM),
            pl.BlockSpec(memory_space=pltpu.SMEM),
        ],
        out_specs=pl.BlockSpec(memory_space=pl.ANY),
        scratch_shapes=[
            pltpu.VMEM((N_DEV, half, k), jnp.float8_e4m3fn),
            pltpu.VMEM((N_DEV, half, k), jnp.float8_e4m3fn),
            pltpu.VMEM((sub, k), jnp.float32),
            pltpu.VMEM((k, n_per), jnp.float32),
            pltpu.VMEM((k, n_per), jnp.bfloat16),
            pltpu.VMEM((2, half, n_per), jnp.float32),
            pltpu.SemaphoreType.DMA,
            pltpu.SemaphoreType.DMA,
            pltpu.SemaphoreType.DMA((2,)),
        ] + [
            pltpu.SemaphoreType.DMA((N_HOP,))
            for _ in range(4 * N_SUB)
        ],
        compiler_params=pltpu.CompilerParams(
            collective_id=0,
            vmem_limit_bytes=100 * 1024 * 1024,
        ),
    )(x, w_mat, scale_x.reshape(1, 1), scale_w.reshape(1, 1))


# device time: 83669 ns/iter; 1.0639x vs baseline; 1.0639x over previous
import jax
import jax.numpy as jnp
from jax import lax
from jax.experimental import pallas as pl
from jax.experimental.pallas import tpu as pltpu

N_DEV = 4
N_HOP = N_DEV - 1
N_SUB = 4


def kernel(x, w_mat, scale_x, scale_w):
    m_per, k = x.shape
    _, n_per = w_mat.shape
    half = m_per // 2
    sub = half // N_SUB

    def body(x_ref, w_ref, sx_ref, sw_ref, out_ref,
             buf_a, buf_b, stage, w32, w16, outblk,
             stage_sem, w_sem, out_sems, *ring_sems):
        my = lax.axis_index("i")
        left = lax.rem(my + N_DEV - 1, N_DEV)
        right = lax.rem(my + 1, N_DEV)

        dirs = {
            "a": dict(buf=buf_a, peer=right, base=0,
                      origin=lambda s: lax.rem(my - s + N_DEV, N_DEV)),
            "b": dict(buf=buf_b, peer=left, base=half,
                      origin=lambda s: lax.rem(my + s, N_DEV)),
        }
        for q in range(N_SUB):
            for di, d in enumerate(("a", "b")):
                dirs[d].setdefault("send", {})[q] = ring_sems[4 * q + 2 * di]
                dirs[d].setdefault("recv", {})[q] = ring_sems[4 * q + 2 * di + 1]

        def make_rdma(d, h, q):
            dd = dirs[d]
            return pltpu.make_async_remote_copy(
                src_ref=dd["buf"].at[h, pl.ds(q * sub, sub), :],
                dst_ref=dd["buf"].at[h + 1, pl.ds(q * sub, sub), :],
                send_sem=dd["send"][q].at[h],
                recv_sem=dd["recv"][q].at[h],
                device_id=(dd["peer"],),
                device_id_type=pl.DeviceIdType.MESH,
            )

        rdmas = {}

        def stage_own(d, q):
            rows = dirs[d]["base"] + q * sub
            cp = pltpu.make_async_copy(
                x_ref.at[pl.ds(rows, sub), :], stage, stage_sem)
            cp.start()
            cp.wait()
            dirs[d]["buf"][0, pl.ds(q * sub, sub), :] = (
                stage[...].astype(jnp.float8_e4m3fn))

        for d in ("a", "b"):
            stage_own(d, 0)

        barrier = pltpu.get_barrier_semaphore()
        for nbr in (left, right):
            pl.semaphore_signal(
                barrier, inc=1,
                device_id=(nbr,), device_id_type=pl.DeviceIdType.MESH,
            )
        pl.semaphore_wait(barrier, 2)

        for d in ("a", "b"):
            r = make_rdma(d, 0, 0)
            r.start()
            rdmas[(d, 0, 0)] = r

        for q in range(1, N_SUB):
            for d in ("a", "b"):
                stage_own(d, q)
                r = make_rdma(d, 0, q)
                r.start()
                rdmas[(d, 0, q)] = r

        cp_w = pltpu.make_async_copy(w_ref, w32, w_sem)
        cp_w.start()
        cp_w.wait()
        w16[...] = w32[...].astype(jnp.bfloat16)

        scale = sx_ref[0, 0] * sw_ref[0, 0]
        out_copies = []

        def compute_block(d, s, row0, nrows):
            dd = dirs[d]
            blk = len(out_copies)
            slot = blk % 2
            if blk >= 2:
                out_copies[blk - 2].wait()
            acc = jnp.dot(
                dd["buf"][s, pl.ds(row0, nrows), :].astype(jnp.bfloat16),
                w16[...], preferred_element_type=jnp.float32)
            outblk[slot, pl.ds(0, nrows), :] = jnp.maximum(acc * scale, 0.0)
            base = dd["origin"](s) * m_per + dd["base"]
            cp = pltpu.make_async_copy(
                outblk.at[slot, pl.ds(0, nrows), :],
                out_ref.at[pl.ds(base + row0, nrows), :],
                out_sems.at[slot],
            )
            cp.start()
            out_copies.append(cp)

        def compute_slot(s):
            compute_block("a", s, 0, half)
            compute_block("b", s, 0, half)

        compute_slot(0)

        for h in range(1, N_HOP):
            for q in range(N_SUB):
                for d in ("a", "b"):
                    rdmas[(d, h - 1, q)].wait()
                    r = make_rdma(d, h, q)
                    r.start()
                    rdmas[(d, h, q)] = r
            compute_slot(h)

        for q in range(N_SUB):
            for d in ("a", "b"):
                rdmas[(d, N_HOP - 1, q)].wait()
            for d in ("a", "b"):
                compute_block(d, N_DEV - 1, q * sub, sub)

        out_copies[-2].wait()
        out_copies[-1].wait()

    return pl.pallas_call(
        body,
        out_shape=jax.ShapeDtypeStruct((N_DEV * m_per, n_per), jnp.float32),
        in_specs=[
            pl.BlockSpec(memory_space=pl.ANY),
            pl.BlockSpec(memory_space=pl.ANY),
            pl.BlockSpec(memory_space=pltpu.SMEM),
            pl.BlockSpec(memory_space=pltpu.SMEM),
        ],
        out_specs=pl.BlockSpec(memory_space=pl.ANY),
        scratch_shapes=[
            pltpu.VMEM((N_DEV, half, k), jnp.float8_e4m3fn),
            pltpu.VMEM((N_DEV, half, k), jnp.float8_e4m3fn),
            pltpu.VMEM((sub, k), jnp.float32),
            pltpu.VMEM((k, n_per), jnp.float32),
            pltpu.VMEM((k, n_per), jnp.bfloat16),
            pltpu.VMEM((2, half, n_per), jnp.float32),
            pltpu.SemaphoreType.DMA,
            pltpu.SemaphoreType.DMA,
            pltpu.SemaphoreType.DMA((2,)),
        ] + [
            pltpu.SemaphoreType.DMA((N_HOP,))
            for _ in range(4 * N_SUB)
        ],
        compiler_params=pltpu.CompilerParams(
            collective_id=0,
            vmem_limit_bytes=100 * 1024 * 1024,
        ),
    )(x, w_mat, scale_x.reshape(1, 1), scale_w.reshape(1, 1))
